# baseline (speedup 1.0000x reference)
import functools

import jax
import jax.numpy as jnp
from jax import lax
from jax.experimental import pallas as pl
from jax.experimental.pallas import tpu as pltpu
from jax.experimental.pallas import tpu_sc as plsc

HIDDEN = 2048
NUM_CB = 32
L = 16
NWORK = 32
TOK = 4096
TPW = TOK // NWORK
GRP = 8
NGRP = TPW // GRP
NHID = HIDDEN // L
UR = 16
UPT = NUM_CB // UR
NUNIT = TPW * UPT
NBUF = 3
TBLK = 3
NBLK = TPW // TBLK


def _tree_sum(vals):
    while len(vals) > 1:
        vals = [a + b for a, b in zip(vals[::2], vals[1::2])] \
            + ([vals[-1]] if len(vals) % 2 else [])
    return vals[0]


def _sc_embed(ids_audio, ids_text, text_table, audio_table, offsets):
    mesh = plsc.VectorSubcoreMesh(core_axis_name="c", subcore_axis_name="s")

    @functools.partial(
        pl.kernel,
        mesh=mesh,
        out_type=jax.ShapeDtypeStruct((TOK, HIDDEN), jnp.float32),
        scratch_types=[
            pltpu.VMEM((TPW * NUM_CB,), jnp.int32),
            pltpu.VMEM((TPW,), jnp.int32),
            pltpu.VMEM((NUM_CB,), jnp.int32),
            pltpu.VMEM((GRP, HIDDEN), jnp.float32),
            pltpu.VMEM((UR, HIDDEN), jnp.float32),
            pltpu.VMEM((UR, HIDDEN), jnp.float32),
            pltpu.VMEM((UR, HIDDEN), jnp.float32),
            pltpu.VMEM((2, 1, HIDDEN), jnp.float32),
            pltpu.SemaphoreType.DMA,
            pltpu.SemaphoreType.DMA,
            pltpu.SemaphoreType.DMA,
            pltpu.SemaphoreType.DMA,
            pltpu.SemaphoreType.DMA,
            pltpu.SemaphoreType.DMA,
        ],
    )
    def body(ids_audio_h, ids_text_h, ttab_h, atab_h, offs_h, out_h,
             aidx_v, tid_v, offs_v, tb, ab0, ab1, ab2, acc,
             sem_a0, sem_a1, sem_a2, sem_t, sem_o0, sem_o1):
        wid = lax.axis_index("s") * 2 + lax.axis_index("c")
        base = wid * TPW
        abufs = (ab0, ab1, ab2)
        asems = (sem_a0, sem_a1, sem_a2)

        pltpu.sync_copy(ids_audio_h.at[pl.ds(base * NUM_CB, TPW * NUM_CB)], aidx_v)
        pltpu.sync_copy(ids_text_h.at[pl.ds(base, TPW)], tid_v)
        pltpu.sync_copy(offs_h, offs_v)

        pltpu.async_copy(ttab_h.at[tid_v.at[pl.ds(0, GRP)]], tb, sem_t)

        zeros = jnp.zeros((L,), jnp.int32)
        offs01 = offs_v[pl.ds(0, L)]
        offs23 = offs_v[pl.ds(L, L)]

        def cidx(t, carry):
            tok01 = aidx_v[pl.ds(NUM_CB * t, L)]
            tok23 = aidx_v[pl.ds(NUM_CB * t + L, L)]
            aidx_v[pl.ds(NUM_CB * t, L)] = jnp.where(tok01 == 0, zeros, tok01 + offs01)
            aidx_v[pl.ds(NUM_CB * t + L, L)] = jnp.where(tok23 == 0, zeros, tok23 + offs23)
            return carry
        lax.fori_loop(0, TPW, cidx, 0)

        pltpu.async_copy(atab_h.at[aidx_v.at[pl.ds(0, UR)]], ab0, sem_a0)
        pltpu.async_copy(atab_h.at[aidx_v.at[pl.ds(UR, UR)]], ab1, sem_a1)

        def token_step(t, j, fire_ok=(True, True)):
            g = t // GRP
            gl = t % GRP
            po = t % 2

            @pl.when(gl == 0)
            def _():
                pltpu.make_async_copy(ttab_h.at[pl.ds(0, GRP)], tb, sem_t).wait()

            @pl.when(jnp.logical_and(po == 0, t >= 2))
            def _():
                pltpu.make_async_copy(out_h.at[pl.ds(0, 1)], acc.at[0], sem_o0).wait()

            @pl.when(jnp.logical_and(po == 1, t >= 2))
            def _():
                pltpu.make_async_copy(out_h.at[pl.ds(0, 1)], acc.at[1], sem_o1).wait()

            bA = abufs[(UPT * j) % NBUF]
            sA = asems[(UPT * j) % NBUF]
            bB = abufs[(UPT * j + 1) % NBUF]
            sB = asems[(UPT * j + 1) % NBUF]
            if fire_ok[0]:
                pltpu.async_copy(
                    atab_h.at[aidx_v.at[pl.ds((UPT * t + 2) * UR, UR)]],
                    abufs[(UPT * j + 2) % NBUF], asems[(UPT * j + 2) % NBUF])
            pltpu.make_async_copy(atab_h.at[pl.ds(0, UR)], bA, sA).wait()
            pltpu.make_async_copy(atab_h.at[pl.ds(0, UR)], bB, sB).wait()

            def acc_pass(c, carry2, bA=bA, bB=bB):
                for k in range(2):
                    cs = pl.ds((2 * c + k) * L, L)
                    vals = ([tb[gl, cs]]
                            + [bA[r, cs] for r in range(UR)]
                            + [bB[r, cs] for r in range(UR)])
                    acc[po, 0, cs] = _tree_sum(vals)
                return carry2
            lax.fori_loop(0, NHID // 2, acc_pass, 0)

            @pl.when(jnp.logical_and(gl == GRP - 1, g + 1 < NGRP))
            def _():
                pltpu.async_copy(
                    ttab_h.at[tid_v.at[pl.ds((g + 1) * GRP, GRP)]], tb, sem_t)

            if fire_ok[1]:
                pltpu.async_copy(
                    atab_h.at[aidx_v.at[pl.ds((UPT * t + 3) * UR, UR)]],
                    abufs[(UPT * j) % NBUF], asems[(UPT * j) % NBUF])

            @pl.when(po == 0)
            def _():
                pltpu.async_copy(acc.at[0], out_h.at[pl.ds(base + t, 1)], sem_o0)

            @pl.when(po == 1)
            def _():
                pltpu.async_copy(acc.at[1], out_h.at[pl.ds(base + t, 1)], sem_o1)

        def blk_body(b, carry):
            for j in range(TBLK):
                token_step(TBLK * b + j, j)
            return carry
        lax.fori_loop(0, NBLK, blk_body, 0)

        token_step(jnp.int32(TBLK * NBLK), 0, fire_ok=(True, True))
        token_step(jnp.int32(TBLK * NBLK + 1), 1, fire_ok=(False, False))

        pltpu.make_async_copy(out_h.at[pl.ds(0, 1)], acc.at[0], sem_o0).wait()
        pltpu.make_async_copy(out_h.at[pl.ds(0, 1)], acc.at[1], sem_o1).wait()

    return body(ids_audio, ids_text, text_table, audio_table, offsets)


def kernel(input_ids, text_table, audio_table, audio_tokens_offsets):
    b, s, _ = input_ids.shape
    ids = input_ids.reshape(b * s, NUM_CB + 1).astype(jnp.int32)
    ids_audio = ids[:, :NUM_CB].reshape(TOK * NUM_CB)
    ids_text = ids[:, NUM_CB]
    offs = audio_tokens_offsets.astype(jnp.int32)
    out = _sc_embed(ids_audio, ids_text, text_table, audio_table, offs)
    return out.reshape(b, s, HIDDEN)


# speedup vs baseline: 2.1150x; 2.1150x over previous
import functools

import jax
import jax.numpy as jnp
from jax import lax
from jax.experimental import pallas as pl
from jax.experimental.pallas import tpu as pltpu
from jax.experimental.pallas import tpu_sc as plsc

HIDDEN = 2048
NUM_CB = 32
L = 16
NWORK = 32
TOK = 4096
TPW = TOK // NWORK
GRP = 8
NGRP = TPW // GRP
NHID = HIDDEN // L
UR = 16
UPT = NUM_CB // UR
NUNIT = TPW * UPT
NBUF = 3
TBLK = 3
NBLK = TPW // TBLK


def _tree_sum(vals):
    while len(vals) > 1:
        vals = [a + b for a, b in zip(vals[::2], vals[1::2])] \
            + ([vals[-1]] if len(vals) % 2 else [])
    return vals[0]


def _sc_embed(ids_audio, ids_text, text_table, audio_table, offsets):
    mesh = plsc.VectorSubcoreMesh(core_axis_name="c", subcore_axis_name="s")

    @functools.partial(
        pl.kernel,
        mesh=mesh,
        out_type=jax.ShapeDtypeStruct((TOK, HIDDEN), jnp.float32),
        scratch_types=[
            pltpu.VMEM((TPW * NUM_CB,), jnp.int32),
            pltpu.VMEM((TPW,), jnp.int32),
            pltpu.VMEM((NUM_CB,), jnp.int32),
            pltpu.VMEM((GRP, HIDDEN), jnp.float32),
            pltpu.VMEM((UR, HIDDEN), jnp.float32),
            pltpu.VMEM((UR, HIDDEN), jnp.float32),
            pltpu.VMEM((UR, HIDDEN), jnp.float32),
            pltpu.VMEM((2, 1, HIDDEN), jnp.float32),
            pltpu.SemaphoreType.DMA,
            pltpu.SemaphoreType.DMA,
            pltpu.SemaphoreType.DMA,
            pltpu.SemaphoreType.DMA,
            pltpu.SemaphoreType.DMA,
            pltpu.SemaphoreType.DMA,
        ],
    )
    def body(ids_audio_h, ids_text_h, ttab_h, atab_h, offs_h, out_h,
             aidx_v, tid_v, offs_v, tb, ab0, ab1, ab2, acc,
             sem_a0, sem_a1, sem_a2, sem_t, sem_o0, sem_o1):
        wid = lax.axis_index("s") * 2 + lax.axis_index("c")
        base = wid * TPW
        abufs = (ab0, ab1, ab2)
        asems = (sem_a0, sem_a1, sem_a2)

        pltpu.sync_copy(ids_audio_h.at[pl.ds(base * NUM_CB, TPW * NUM_CB)], aidx_v)
        pltpu.sync_copy(ids_text_h.at[pl.ds(base, TPW)], tid_v)
        pltpu.sync_copy(offs_h, offs_v)

        pltpu.async_copy(ttab_h.at[tid_v.at[pl.ds(0, GRP)]], tb, sem_t)

        zeros = jnp.zeros((L,), jnp.int32)
        offs01 = offs_v[pl.ds(0, L)]
        offs23 = offs_v[pl.ds(L, L)]

        def cidx(t, carry):
            tok01 = aidx_v[pl.ds(NUM_CB * t, L)]
            tok23 = aidx_v[pl.ds(NUM_CB * t + L, L)]
            aidx_v[pl.ds(NUM_CB * t, L)] = jnp.where(tok01 == 0, zeros, tok01 + offs01)
            aidx_v[pl.ds(NUM_CB * t + L, L)] = jnp.where(tok23 == 0, zeros, tok23 + offs23)
            return carry
        lax.fori_loop(0, TPW, cidx, 0)

        pltpu.async_copy(atab_h.at[aidx_v.at[pl.ds(0, UR)]], ab0, sem_a0)
        pltpu.async_copy(atab_h.at[aidx_v.at[pl.ds(UR, UR)]], ab1, sem_a1)

        def token_step(t, j, fire_ok=(True, True)):
            g = t // GRP
            gl = t % GRP
            po = t % 2

            @pl.when(gl == 0)
            def _():
                pltpu.make_async_copy(ttab_h.at[pl.ds(0, GRP)], tb, sem_t).wait()

            @pl.when(jnp.logical_and(po == 0, t >= 2))
            def _():
                pltpu.make_async_copy(out_h.at[pl.ds(0, 1)], acc.at[0], sem_o0).wait()

            @pl.when(jnp.logical_and(po == 1, t >= 2))
            def _():
                pltpu.make_async_copy(out_h.at[pl.ds(0, 1)], acc.at[1], sem_o1).wait()

            for h in range(UPT):
                cur = abufs[(UPT * j + h) % NBUF]
                cur_s = asems[(UPT * j + h) % NBUF]
                if fire_ok[h]:
                    nxt = abufs[(UPT * j + h + 2) % NBUF]
                    nxt_s = asems[(UPT * j + h + 2) % NBUF]
                    pltpu.async_copy(
                        atab_h.at[aidx_v.at[pl.ds((UPT * t + h + 2) * UR, UR)]],
                        nxt, nxt_s)
                pltpu.make_async_copy(atab_h.at[pl.ds(0, UR)], cur, cur_s).wait()

                if h == 0:
                    @plsc.parallel_loop(0, NHID // 2, step=1, unroll=2)
                    def _(c, cur=cur):
                        for k in range(2):
                            cs = pl.ds((2 * c + k) * L, L)
                            vals = [tb[gl, cs]] + [cur[r, cs] for r in range(UR)]
                            acc[po, 0, cs] = _tree_sum(vals)

                    @pl.when(jnp.logical_and(gl == GRP - 1, g + 1 < NGRP))
                    def _():
                        pltpu.async_copy(
                            ttab_h.at[tid_v.at[pl.ds((g + 1) * GRP, GRP)]], tb, sem_t)
                else:
                    @plsc.parallel_loop(0, NHID // 2, step=1, unroll=2)
                    def _(c, cur=cur):
                        for k in range(2):
                            cs = pl.ds((2 * c + k) * L, L)
                            vals = [acc[po, 0, cs]] + [cur[r, cs] for r in range(UR)]
                            acc[po, 0, cs] = _tree_sum(vals)

            @pl.when(po == 0)
            def _():
                pltpu.async_copy(acc.at[0], out_h.at[pl.ds(base + t, 1)], sem_o0)

            @pl.when(po == 1)
            def _():
                pltpu.async_copy(acc.at[1], out_h.at[pl.ds(base + t, 1)], sem_o1)

        def blk_body(b, carry):
            for j in range(TBLK):
                token_step(TBLK * b + j, j)
            return carry
        lax.fori_loop(0, NBLK, blk_body, 0)

        token_step(jnp.int32(TBLK * NBLK), 0, fire_ok=(True, True))
        token_step(jnp.int32(TBLK * NBLK + 1), 1, fire_ok=(False, False))

        pltpu.make_async_copy(out_h.at[pl.ds(0, 1)], acc.at[0], sem_o0).wait()
        pltpu.make_async_copy(out_h.at[pl.ds(0, 1)], acc.at[1], sem_o1).wait()

    return body(ids_audio, ids_text, text_table, audio_table, offsets)


def kernel(input_ids, text_table, audio_table, audio_tokens_offsets):
    b, s, _ = input_ids.shape
    ids = input_ids.reshape(b * s, NUM_CB + 1).astype(jnp.int32)
    ids_audio = ids[:, :NUM_CB].reshape(TOK * NUM_CB)
    ids_text = ids[:, NUM_CB]
    offs = audio_tokens_offsets.astype(jnp.int32)
    out = _sc_embed(ids_audio, ids_text, text_table, audio_table, offs)
    return out.reshape(b, s, HIDDEN)


# speedup vs baseline: 2.1832x; 1.0323x over previous
import functools

import jax
import jax.numpy as jnp
from jax import lax
from jax.experimental import pallas as pl
from jax.experimental.pallas import tpu as pltpu
from jax.experimental.pallas import tpu_sc as plsc

HIDDEN = 2048
NUM_CB = 32
L = 16
NWORK = 32
TOK = 4096
TPW = TOK // NWORK
GRP = 8
NGRP = TPW // GRP
NHID = HIDDEN // L
UR = 16
UPT = NUM_CB // UR
NUNIT = TPW * UPT
NBUF = 3
TBLK = 3
NBLK = TPW // TBLK


def _tree_sum(vals):
    while len(vals) > 1:
        vals = [a + b for a, b in zip(vals[::2], vals[1::2])] \
            + ([vals[-1]] if len(vals) % 2 else [])
    return vals[0]


def _sc_embed(ids_audio, ids_text, text_table, audio_table, offsets):
    mesh = plsc.VectorSubcoreMesh(core_axis_name="c", subcore_axis_name="s")

    @functools.partial(
        pl.kernel,
        mesh=mesh,
        out_type=jax.ShapeDtypeStruct((TOK, HIDDEN), jnp.float32),
        scratch_types=[
            pltpu.VMEM((TPW * NUM_CB,), jnp.int32),
            pltpu.VMEM((TPW,), jnp.int32),
            pltpu.VMEM((NUM_CB,), jnp.int32),
            pltpu.VMEM((GRP, HIDDEN), jnp.float32),
            pltpu.VMEM((UR, HIDDEN), jnp.float32),
            pltpu.VMEM((UR, HIDDEN), jnp.float32),
            pltpu.VMEM((UR, HIDDEN), jnp.float32),
            pltpu.VMEM((2, 1, HIDDEN), jnp.float32),
            pltpu.SemaphoreType.DMA,
            pltpu.SemaphoreType.DMA,
            pltpu.SemaphoreType.DMA,
            pltpu.SemaphoreType.DMA,
            pltpu.SemaphoreType.DMA,
            pltpu.SemaphoreType.DMA,
        ],
    )
    def body(ids_audio_h, ids_text_h, ttab_h, atab_h, offs_h, out_h,
             aidx_v, tid_v, offs_v, tb, ab0, ab1, ab2, acc,
             sem_a0, sem_a1, sem_a2, sem_t, sem_o0, sem_o1):
        wid = lax.axis_index("s") * 2 + lax.axis_index("c")
        base = wid * TPW
        abufs = (ab0, ab1, ab2)
        asems = (sem_a0, sem_a1, sem_a2)

        pltpu.sync_copy(ids_audio_h.at[pl.ds(base * NUM_CB, TPW * NUM_CB)], aidx_v)
        pltpu.sync_copy(ids_text_h.at[pl.ds(base, TPW)], tid_v)
        pltpu.sync_copy(offs_h, offs_v)

        pltpu.async_copy(ttab_h.at[tid_v.at[pl.ds(0, GRP)]], tb, sem_t)

        zeros = jnp.zeros((L,), jnp.int32)
        offs01 = offs_v[pl.ds(0, L)]
        offs23 = offs_v[pl.ds(L, L)]

        def cidx(t, carry):
            tok01 = aidx_v[pl.ds(NUM_CB * t, L)]
            tok23 = aidx_v[pl.ds(NUM_CB * t + L, L)]
            aidx_v[pl.ds(NUM_CB * t, L)] = jnp.where(tok01 == 0, zeros, tok01 + offs01)
            aidx_v[pl.ds(NUM_CB * t + L, L)] = jnp.where(tok23 == 0, zeros, tok23 + offs23)
            return carry
        lax.fori_loop(0, TPW, cidx, 0)

        pltpu.async_copy(atab_h.at[aidx_v.at[pl.ds(0, UR)]], ab0, sem_a0)
        pltpu.async_copy(atab_h.at[aidx_v.at[pl.ds(UR, UR)]], ab1, sem_a1)

        def token_step(t, j, fire_ok=(True, True)):
            g = t // GRP
            gl = t % GRP
            po = t % 2

            @pl.when(gl == 0)
            def _():
                pltpu.make_async_copy(ttab_h.at[pl.ds(0, GRP)], tb, sem_t).wait()

            @pl.when(jnp.logical_and(po == 0, t >= 2))
            def _():
                pltpu.make_async_copy(out_h.at[pl.ds(0, 1)], acc.at[0], sem_o0).wait()

            @pl.when(jnp.logical_and(po == 1, t >= 2))
            def _():
                pltpu.make_async_copy(out_h.at[pl.ds(0, 1)], acc.at[1], sem_o1).wait()

            for h in range(UPT):
                cur = abufs[(UPT * j + h) % NBUF]
                cur_s = asems[(UPT * j + h) % NBUF]
                if fire_ok[h]:
                    nxt = abufs[(UPT * j + h + 2) % NBUF]
                    nxt_s = asems[(UPT * j + h + 2) % NBUF]
                    pltpu.async_copy(
                        atab_h.at[aidx_v.at[pl.ds((UPT * t + h + 2) * UR, UR)]],
                        nxt, nxt_s)
                pltpu.make_async_copy(atab_h.at[pl.ds(0, UR)], cur, cur_s).wait()

                if h == 0:
                    @plsc.parallel_loop(0, NHID // 2, step=1, unroll=4)
                    def _(c, cur=cur):
                        for k in range(2):
                            cs = pl.ds((2 * c + k) * L, L)
                            vals = [tb[gl, cs]] + [cur[r, cs] for r in range(UR)]
                            acc[po, 0, cs] = _tree_sum(vals)

                    @pl.when(jnp.logical_and(gl == GRP - 1, g + 1 < NGRP))
                    def _():
                        pltpu.async_copy(
                            ttab_h.at[tid_v.at[pl.ds((g + 1) * GRP, GRP)]], tb, sem_t)
                else:
                    @plsc.parallel_loop(0, NHID // 2, step=1, unroll=4)
                    def _(c, cur=cur):
                        for k in range(2):
                            cs = pl.ds((2 * c + k) * L, L)
                            vals = [acc[po, 0, cs]] + [cur[r, cs] for r in range(UR)]
                            acc[po, 0, cs] = _tree_sum(vals)

            @pl.when(po == 0)
            def _():
                pltpu.async_copy(acc.at[0], out_h.at[pl.ds(base + t, 1)], sem_o0)

            @pl.when(po == 1)
            def _():
                pltpu.async_copy(acc.at[1], out_h.at[pl.ds(base + t, 1)], sem_o1)

        def blk_body(b, carry):
            for j in range(TBLK):
                token_step(TBLK * b + j, j)
            return carry
        lax.fori_loop(0, NBLK, blk_body, 0)

        token_step(jnp.int32(TBLK * NBLK), 0, fire_ok=(True, True))
        token_step(jnp.int32(TBLK * NBLK + 1), 1, fire_ok=(False, False))

        pltpu.make_async_copy(out_h.at[pl.ds(0, 1)], acc.at[0], sem_o0).wait()
        pltpu.make_async_copy(out_h.at[pl.ds(0, 1)], acc.at[1], sem_o1).wait()

    return body(ids_audio, ids_text, text_table, audio_table, offsets)


def kernel(input_ids, text_table, audio_table, audio_tokens_offsets):
    b, s, _ = input_ids.shape
    ids = input_ids.reshape(b * s, NUM_CB + 1).astype(jnp.int32)
    ids_audio = ids[:, :NUM_CB].reshape(TOK * NUM_CB)
    ids_text = ids[:, NUM_CB]
    offs = audio_tokens_offsets.astype(jnp.int32)
    out = _sc_embed(ids_audio, ids_text, text_table, audio_table, offs)
    return out.reshape(b, s, HIDDEN)
